# baseline (speedup 1.0000x reference)
import functools

import jax
import jax.numpy as jnp
from jax import lax
from jax.experimental import pallas as pl
from jax.experimental.pallas import tpu as pltpu
from jax.experimental.pallas import tpu_sc as plsc

NC = 2
NS = 16
L = 16
NW = NC * NS
CHUNK = 128
DEPTH = 4
DC = DEPTH * CHUNK
NPH = 4
D = 128
DH = D // NC

_mesh = functools.partial(
    plsc.VectorSubcoreMesh,
    core_axis_name="c", subcore_axis_name="s", num_cores=NC, num_subcores=NS,
)

_SC_PARAMS = pltpu.CompilerParams(needs_layout_passes=False,
                                  use_tc_tiling_on_sc=False)


def _make_deg_kernel(n_pad, e_w):
    @functools.partial(
        pl.kernel,
        out_type=jax.ShapeDtypeStruct((NW, n_pad), jnp.float32),
        mesh=_mesh(),
        compiler_params=_SC_PARAMS,
        scratch_types=[
            pltpu.VMEM((n_pad,), jnp.float32),
            pltpu.VMEM((e_w,), jnp.int32),
            pltpu.VMEM((e_w,), jnp.float32),
            pltpu.SemaphoreType.DMA,
        ],
    )
    def deg_kernel(dst_hbm, ew_hbm, out_hbm, deg_v, dst_v, ew_v, sem):
        cid = lax.axis_index("c")
        sid = lax.axis_index("s")
        wid = sid * NC + cid
        base = wid * e_w

        ld_d = pltpu.async_copy(dst_hbm.at[pl.ds(base, e_w)], dst_v, sem)
        ld_w = pltpu.async_copy(ew_hbm.at[pl.ds(base, e_w)], ew_v, sem)

        zero = jnp.zeros((L,), jnp.float32)

        def zero_body(i, _):
            deg_v[pl.ds(i * L, L)] = zero
            return 0
        lax.fori_loop(0, n_pad // L, zero_body, 0)
        ld_d.wait()
        ld_w.wait()

        def edge_body(g, _):
            idx = dst_v[pl.ds(g * L, L)]
            val = ew_v[pl.ds(g * L, L)]
            plsc.addupdate_scatter(deg_v, [idx], val)
            return 0
        lax.fori_loop(0, e_w // L, edge_body, 0)

        pltpu.sync_copy(deg_v, out_hbm.at[wid])

    return deg_kernel


def _make_agg_kernel(n_pad, e_w):
    n_chunks = e_w // CHUNK
    n_super = n_chunks // DEPTH
    NB = n_super // NPH
    stripe = n_pad // NS

    scratch = [
        pltpu.VMEM((NPH * DC,), jnp.int32),
        pltpu.VMEM((NPH, DEPTH, CHUNK), jnp.int32),
        pltpu.VMEM((NPH * DC,), jnp.float32),
    ] + [pltpu.VMEM((CHUNK, DH), jnp.float32) for _ in range(DEPTH)
    ] + [
        pltpu.VMEM_SHARED((n_pad, DH), jnp.float32),
        pltpu.VMEM_SHARED((n_pad, DH), jnp.float32),
        pltpu.SemaphoreType.DMA,
    ] + [pltpu.SemaphoreType.DMA for _ in range(NPH)
    ] + [pltpu.SemaphoreType.DMA for _ in range(2 * DEPTH)]

    @functools.partial(
        pl.kernel,
        out_type=jax.ShapeDtypeStruct((NC, n_pad, DH), jnp.float32),
        mesh=_mesh(),
        compiler_params=_SC_PARAMS,
        scratch_types=scratch,
    )
    def agg_kernel(h_hbm, src_hbm, dst_hbm, ew_hbm, out_hbm,
                   src_v, dst_v, ew_v, *rest):
        rows = rest[:DEPTH]
        out_sh = rest[DEPTH]
        h_sh = rest[DEPTH + 1]
        ldsem = rest[DEPTH + 2]
        esem = rest[DEPTH + 3:DEPTH + 3 + NPH]
        gsem = rest[DEPTH + 3 + NPH:DEPTH + 3 + NPH + DEPTH]
        ssem = rest[DEPTH + 3 + NPH + DEPTH:]

        cid = lax.axis_index("c")
        sid = lax.axis_index("s")
        base = sid * e_w
        r0 = sid * stripe

        def _stage_copies(g, q):
            return [
                (src_hbm.at[pl.ds(base + g * DC, DC)],
                 src_v.at[pl.ds(q * DC, DC)]),
                (dst_hbm.at[sid, pl.ds(g * DEPTH, DEPTH)], dst_v.at[q]),
                (ew_hbm.at[pl.ds(base + g * DC, DC)],
                 ew_v.at[pl.ds(q * DC, DC)]),
            ]

        def stage(g, q):
            for s, d in _stage_copies(g, q):
                pltpu.async_copy(s, d, esem[q])

        def stage_wait(g, q):
            for s, d in _stage_copies(g, q):
                pltpu.make_async_copy(s, d, esem[q]).wait()

        def scat_wait(q):
            for k in range(DEPTH):
                pltpu.make_async_copy(
                    rows[k], out_sh.at[dst_v.at[q, k]], ssem[k]).wait()

        ld_h = pltpu.async_copy(h_hbm.at[pl.ds(cid * n_pad + r0, stripe)],
                                h_sh.at[pl.ds(r0, stripe)], ldsem)
        for q in range(NPH):
            stage(q, q)

        zero = jnp.zeros((L,), jnp.float32)

        def zb_body(i, _):
            for j in range(DH // L):
                rows[0][i, pl.ds(j * L, L)] = zero
            return 0
        lax.fori_loop(0, CHUNK, zb_body, 0)

        for i in range(stripe // CHUNK):
            pltpu.sync_copy(rows[0], out_sh.at[pl.ds(r0 + i * CHUNK, CHUNK)])

        ld_h.wait()
        plsc.subcore_barrier()

        def group(g, q):
            gathers = []
            for k in range(DEPTH):
                idx = src_v.at[pl.ds(q * DC + k * CHUNK, CHUNK)]
                gathers.append(
                    pltpu.async_copy(h_sh.at[idx], rows[k], gsem[k]))
            for k in range(DEPTH):
                gathers[k].wait()
                b16 = jnp.full((L,), q * DC + k * CHUNK, jnp.int32)

                @plsc.parallel_loop(0, CHUNK, unroll=8)
                def row_body(r):
                    r16 = lax.broadcast_in_dim(r, (L,), ())
                    nvec = plsc.load_gather(ew_v, [b16 + r16])
                    for j in range(DH // L):
                        rows[k][r, pl.ds(j * L, L)] = (
                            rows[k][r, pl.ds(j * L, L)] * nvec)

                pltpu.make_async_copy(
                    rows[k], out_sh.at[dst_v.at[q, k]], ssem[k]).start(add=True)

        def body(i, first=False, last=False):
            g = NPH * i
            for q in range(NPH):
                stage_wait(g + q, q)
                if not (first and q == 0):
                    scat_wait((q - 1) % NPH)
                    if (not first or q > 0) and (not last or q == 0):
                        stage(g + q + NPH - 1, (q - 1) % NPH)
                group(g + q, q)

        body(0, first=True)

        def loop_body(i, _):
            body(i)
            return 0
        lax.fori_loop(1, NB - 1, loop_body, 0)

        body(NB - 1, last=True)
        scat_wait(NPH - 1)

        plsc.subcore_barrier()
        pltpu.sync_copy(out_sh.at[pl.ds(r0, stripe)],
                        out_hbm.at[cid, pl.ds(r0, stripe)])

    return agg_kernel


def _tc_matmul(x, w, n_pad):
    n = x.shape[0]

    def mm(x_ref, w_ref, o_ref):
        h = jnp.dot(x_ref[...], w_ref[...], preferred_element_type=jnp.float32)
        o_ref[0, 0:n, :] = h[:, 0:DH]
        o_ref[1, 0:n, :] = h[:, DH:D]
    return pl.pallas_call(
        mm, out_shape=jax.ShapeDtypeStruct((NC, n_pad, DH), jnp.float32))(x, w)


def _tc_dish(deg_p, h1):
    n_pad = deg_p.shape[1]

    def body(dp_ref, h_ref, dis_ref, hs_ref):
        deg = jnp.sum(dp_ref[...], axis=0)
        dis = jnp.where(deg > 0, lax.rsqrt(jnp.maximum(deg, 1e-12)), 0.0)
        dis_ref[0, :] = dis
        hs_ref[0, :, :] = h_ref[0] * dis[:, None]
        hs_ref[1, :, :] = h_ref[1] * dis[:, None]
    return pl.pallas_call(
        body, out_shape=(
            jax.ShapeDtypeStruct((1, n_pad), jnp.float32),
            jax.ShapeDtypeStruct((NC, n_pad, DH), jnp.float32),
        ))(deg_p, h1)


def _tc_mid(parts, dis, b1, w2):
    def body(p_ref, dis_ref, b_ref, w_ref, o_ref):
        d = dis_ref[0][:, None]
        s = jnp.concatenate([p_ref[0], p_ref[1]], axis=-1) * d + b_ref[...]
        h = jnp.dot(jnp.maximum(s, 0.0), w_ref[...],
                    preferred_element_type=jnp.float32)
        o_ref[0, :, :] = h[:, 0:DH] * d
        o_ref[1, :, :] = h[:, DH:D] * d
    n_pad = parts.shape[1]
    return pl.pallas_call(
        body, out_shape=jax.ShapeDtypeStruct((NC, n_pad, DH),
                                             jnp.float32))(parts, dis, b1, w2)


def _tc_final(parts, dis, b2, gamma, beta, n):
    def body(p_ref, dis_ref, b_ref, g_ref, bt_ref, o_ref):
        d = dis_ref[0][0:n, None]
        s = jnp.concatenate([p_ref[0, 0:n, :], p_ref[1, 0:n, :]],
                            axis=-1) * d + b_ref[...]
        mu = jnp.mean(s, axis=-1, keepdims=True)
        var = jnp.mean((s - mu) ** 2, axis=-1, keepdims=True)
        o_ref[...] = ((s - mu) * lax.rsqrt(var + 1e-5) * g_ref[...]
                      + bt_ref[...])
    return pl.pallas_call(
        body, out_shape=jax.ShapeDtypeStruct((n, D), jnp.float32))(
            parts, dis, b2, gamma, beta)


@jax.jit
def kernel(x, edge_index, edge_weight, W1, b1, W2, b2, gamma, beta):
    n = x.shape[0]
    e = edge_weight.shape[0]

    loop = jnp.arange(n, dtype=jnp.int32)
    src = jnp.concatenate([edge_index[0].astype(jnp.int32), loop])
    dst = jnp.concatenate([edge_index[1].astype(jnp.int32), loop])
    ew = jnp.concatenate([edge_weight, jnp.ones((n,), edge_weight.dtype)])

    e_tot = e + n
    grain = NS * CHUNK * DEPTH * NPH
    e_pad = ((e_tot + grain - 1) // grain) * grain
    pad = e_pad - e_tot
    src = jnp.concatenate([src, jnp.zeros((pad,), jnp.int32)])
    dst = jnp.concatenate([dst, jnp.zeros((pad,), jnp.int32)])
    ew = jnp.concatenate([ew, jnp.zeros((pad,), jnp.float32)])
    e_w = e_pad // NS
    e_w_deg = e_pad // NW
    n_chunks = e_w // CHUNK

    ngrain = NS * L
    n_pad = ((n + ngrain - 1) // ngrain) * ngrain

    dst3 = dst.reshape(NS, n_chunks, CHUNK)

    deg_p = _make_deg_kernel(n_pad, e_w_deg)(dst, ew)
    h1 = _tc_matmul(x, W1, n_pad)
    dis, h1s = _tc_dish(deg_p, h1)

    agg = _make_agg_kernel(n_pad, e_w)
    p1 = agg(h1s.reshape(NC * n_pad, DH), src, dst3, ew)
    h2s = _tc_mid(p1, dis, b1, W2)
    p2 = agg(h2s.reshape(NC * n_pad, DH), src, dst3, ew)
    return _tc_final(p2, dis, b2, gamma, beta, n)


# speedup vs baseline: 1.1635x; 1.1635x over previous
import functools

import jax
import jax.numpy as jnp
from jax import lax
from jax.experimental import pallas as pl
from jax.experimental.pallas import tpu as pltpu
from jax.experimental.pallas import tpu_sc as plsc

NC = 2
NS = 16
L = 16
NW = NC * NS
CHUNK = 128
DEPTH = 4
DC = DEPTH * CHUNK
NPH = 4
D = 128
DH = D // NC

_mesh = functools.partial(
    plsc.VectorSubcoreMesh,
    core_axis_name="c", subcore_axis_name="s", num_cores=NC, num_subcores=NS,
)

_SC_PARAMS = pltpu.CompilerParams(needs_layout_passes=False,
                                  use_tc_tiling_on_sc=False)


def _make_deg_kernel(n_pad, e_w):
    @functools.partial(
        pl.kernel,
        out_type=jax.ShapeDtypeStruct((NW, n_pad), jnp.float32),
        mesh=_mesh(),
        compiler_params=_SC_PARAMS,
        scratch_types=[
            pltpu.VMEM((n_pad,), jnp.float32),
            pltpu.VMEM((e_w,), jnp.int32),
            pltpu.VMEM((e_w,), jnp.float32),
            pltpu.SemaphoreType.DMA,
        ],
    )
    def deg_kernel(dst_hbm, ew_hbm, out_hbm, deg_v, dst_v, ew_v, sem):
        cid = lax.axis_index("c")
        sid = lax.axis_index("s")
        wid = sid * NC + cid
        base = wid * e_w

        ld_d = pltpu.async_copy(dst_hbm.at[pl.ds(base, e_w)], dst_v, sem)
        ld_w = pltpu.async_copy(ew_hbm.at[pl.ds(base, e_w)], ew_v, sem)

        zero = jnp.zeros((L,), jnp.float32)

        def zero_body(i, _):
            deg_v[pl.ds(i * L, L)] = zero
            return 0
        lax.fori_loop(0, n_pad // L, zero_body, 0)
        ld_d.wait()
        ld_w.wait()

        def edge_body(g, _):
            idx = dst_v[pl.ds(g * L, L)]
            val = ew_v[pl.ds(g * L, L)]
            plsc.addupdate_scatter(deg_v, [idx], val)
            return 0
        lax.fori_loop(0, e_w // L, edge_body, 0)

        pltpu.sync_copy(deg_v, out_hbm.at[wid])

    return deg_kernel


def _make_agg_kernel(n_pad, e_w):
    n_chunks = e_w // CHUNK
    n_super = n_chunks // DEPTH
    NB = n_super // NPH
    stripe = n_pad // NS

    scratch = [
        pltpu.VMEM((NPH * DC,), jnp.int32),
        pltpu.VMEM((NPH, DEPTH, CHUNK), jnp.int32),
        pltpu.VMEM((NPH * DC,), jnp.float32),
    ] + [pltpu.VMEM((CHUNK, DH), jnp.float32) for _ in range(DEPTH)
    ] + [
        pltpu.VMEM_SHARED((n_pad, DH), jnp.float32),
        pltpu.VMEM_SHARED((n_pad, DH), jnp.float32),
        pltpu.SemaphoreType.DMA,
    ] + [pltpu.SemaphoreType.DMA for _ in range(NPH)
    ] + [pltpu.SemaphoreType.DMA for _ in range(2 * DEPTH)]

    @functools.partial(
        pl.kernel,
        out_type=jax.ShapeDtypeStruct((NC, n_pad, DH), jnp.float32),
        mesh=_mesh(),
        compiler_params=_SC_PARAMS,
        scratch_types=scratch,
    )
    def agg_kernel(h_hbm, src_hbm, dst_hbm, ew_hbm, out_hbm,
                   src_v, dst_v, ew_v, *rest):
        rows = rest[:DEPTH]
        out_sh = rest[DEPTH]
        h_sh = rest[DEPTH + 1]
        ldsem = rest[DEPTH + 2]
        esem = rest[DEPTH + 3:DEPTH + 3 + NPH]
        gsem = rest[DEPTH + 3 + NPH:DEPTH + 3 + NPH + DEPTH]
        ssem = rest[DEPTH + 3 + NPH + DEPTH:]

        cid = lax.axis_index("c")
        sid = lax.axis_index("s")
        base = sid * e_w
        r0 = sid * stripe

        def _stage_copies(g, q):
            return [
                (src_hbm.at[pl.ds(base + g * DC, DC)],
                 src_v.at[pl.ds(q * DC, DC)]),
                (dst_hbm.at[sid, pl.ds(g * DEPTH, DEPTH)], dst_v.at[q]),
                (ew_hbm.at[pl.ds(base + g * DC, DC)],
                 ew_v.at[pl.ds(q * DC, DC)]),
            ]

        def stage(g, q):
            for s, d in _stage_copies(g, q):
                pltpu.async_copy(s, d, esem[q])

        def stage_wait(g, q):
            for s, d in _stage_copies(g, q):
                pltpu.make_async_copy(s, d, esem[q]).wait()

        def scat_wait(q):
            for k in range(DEPTH):
                pltpu.make_async_copy(
                    rows[k], out_sh.at[dst_v.at[q, k]], ssem[k]).wait()

        ld_h = pltpu.async_copy(h_hbm.at[pl.ds(cid * n_pad + r0, stripe)],
                                h_sh.at[pl.ds(r0, stripe)], ldsem)
        for q in range(NPH):
            stage(q, q)

        zero = jnp.zeros((L,), jnp.float32)

        def zb_body(i, _):
            for j in range(DH // L):
                rows[0][i, pl.ds(j * L, L)] = zero
            return 0
        lax.fori_loop(0, CHUNK, zb_body, 0)

        for i in range(stripe // CHUNK):
            pltpu.sync_copy(rows[0], out_sh.at[pl.ds(r0 + i * CHUNK, CHUNK)])

        ld_h.wait()
        plsc.subcore_barrier()

        def group(g, q):
            gathers = []
            for k in range(DEPTH):
                idx = src_v.at[pl.ds(q * DC + k * CHUNK, CHUNK)]
                gathers.append(
                    pltpu.async_copy(h_sh.at[idx], rows[k], gsem[k]))
            for k in range(DEPTH):
                gathers[k].wait()
                b16 = jnp.full((L,), q * DC + k * CHUNK, jnp.int32)

                @plsc.parallel_loop(0, CHUNK, unroll=4)
                def row_body(r):
                    r16 = lax.broadcast_in_dim(r, (L,), ())
                    nvec = plsc.load_gather(ew_v, [b16 + r16])
                    for j in range(DH // L):
                        rows[k][r, pl.ds(j * L, L)] = (
                            rows[k][r, pl.ds(j * L, L)] * nvec)

                pltpu.make_async_copy(
                    rows[k], out_sh.at[dst_v.at[q, k]], ssem[k]).start(add=True)

        def body(i, first=False, last=False):
            g = NPH * i
            for q in range(NPH):
                stage_wait(g + q, q)
                if not (first and q == 0):
                    scat_wait((q - 1) % NPH)
                    if (not first or q > 0) and (not last or q == 0):
                        stage(g + q + NPH - 1, (q - 1) % NPH)
                group(g + q, q)

        body(0, first=True)

        def loop_body(i, _):
            plsc.subcore_barrier()
            body(i)
            return 0
        lax.fori_loop(1, NB - 1, loop_body, 0)

        body(NB - 1, last=True)
        scat_wait(NPH - 1)

        plsc.subcore_barrier()
        pltpu.sync_copy(out_sh.at[pl.ds(r0, stripe)],
                        out_hbm.at[cid, pl.ds(r0, stripe)])

    return agg_kernel


def _tc_matmul(x, w, n_pad):
    n = x.shape[0]

    def mm(x_ref, w_ref, o_ref):
        h = jnp.dot(x_ref[...], w_ref[...], preferred_element_type=jnp.float32)
        o_ref[0, 0:n, :] = h[:, 0:DH]
        o_ref[1, 0:n, :] = h[:, DH:D]
    return pl.pallas_call(
        mm, out_shape=jax.ShapeDtypeStruct((NC, n_pad, DH), jnp.float32))(x, w)


def _tc_dish(deg_p, h1):
    n_pad = deg_p.shape[1]

    def body(dp_ref, h_ref, dis_ref, hs_ref):
        deg = jnp.sum(dp_ref[...], axis=0)
        dis = jnp.where(deg > 0, lax.rsqrt(jnp.maximum(deg, 1e-12)), 0.0)
        dis_ref[0, :] = dis
        hs_ref[0, :, :] = h_ref[0] * dis[:, None]
        hs_ref[1, :, :] = h_ref[1] * dis[:, None]
    return pl.pallas_call(
        body, out_shape=(
            jax.ShapeDtypeStruct((1, n_pad), jnp.float32),
            jax.ShapeDtypeStruct((NC, n_pad, DH), jnp.float32),
        ))(deg_p, h1)


def _tc_mid(parts, dis, b1, w2):
    def body(p_ref, dis_ref, b_ref, w_ref, o_ref):
        d = dis_ref[0][:, None]
        s = jnp.concatenate([p_ref[0], p_ref[1]], axis=-1) * d + b_ref[...]
        h = jnp.dot(jnp.maximum(s, 0.0), w_ref[...],
                    preferred_element_type=jnp.float32)
        o_ref[0, :, :] = h[:, 0:DH] * d
        o_ref[1, :, :] = h[:, DH:D] * d
    n_pad = parts.shape[1]
    return pl.pallas_call(
        body, out_shape=jax.ShapeDtypeStruct((NC, n_pad, DH),
                                             jnp.float32))(parts, dis, b1, w2)


def _tc_final(parts, dis, b2, gamma, beta, n):
    def body(p_ref, dis_ref, b_ref, g_ref, bt_ref, o_ref):
        d = dis_ref[0][0:n, None]
        s = jnp.concatenate([p_ref[0, 0:n, :], p_ref[1, 0:n, :]],
                            axis=-1) * d + b_ref[...]
        mu = jnp.mean(s, axis=-1, keepdims=True)
        var = jnp.mean((s - mu) ** 2, axis=-1, keepdims=True)
        o_ref[...] = ((s - mu) * lax.rsqrt(var + 1e-5) * g_ref[...]
                      + bt_ref[...])
    return pl.pallas_call(
        body, out_shape=jax.ShapeDtypeStruct((n, D), jnp.float32))(
            parts, dis, b2, gamma, beta)


@jax.jit
def kernel(x, edge_index, edge_weight, W1, b1, W2, b2, gamma, beta):
    n = x.shape[0]
    e = edge_weight.shape[0]

    loop = jnp.arange(n, dtype=jnp.int32)
    src = jnp.concatenate([edge_index[0].astype(jnp.int32), loop])
    dst = jnp.concatenate([edge_index[1].astype(jnp.int32), loop])
    ew = jnp.concatenate([edge_weight, jnp.ones((n,), edge_weight.dtype)])

    e_tot = e + n
    grain = NS * CHUNK * DEPTH * NPH
    e_pad = ((e_tot + grain - 1) // grain) * grain
    pad = e_pad - e_tot
    src = jnp.concatenate([src, jnp.zeros((pad,), jnp.int32)])
    dst = jnp.concatenate([dst, jnp.zeros((pad,), jnp.int32)])
    ew = jnp.concatenate([ew, jnp.zeros((pad,), jnp.float32)])
    e_w = e_pad // NS
    e_w_deg = e_pad // NW
    n_chunks = e_w // CHUNK

    ngrain = NS * L
    n_pad = ((n + ngrain - 1) // ngrain) * ngrain

    dst3 = dst.reshape(NS, n_chunks, CHUNK)

    deg_p = _make_deg_kernel(n_pad, e_w_deg)(dst, ew)
    h1 = _tc_matmul(x, W1, n_pad)
    dis, h1s = _tc_dish(deg_p, h1)

    agg = _make_agg_kernel(n_pad, e_w)
    p1 = agg(h1s.reshape(NC * n_pad, DH), src, dst3, ew)
    h2s = _tc_mid(p1, dis, b1, W2)
    p2 = agg(h2s.reshape(NC * n_pad, DH), src, dst3, ew)
    return _tc_final(p2, dis, b2, gamma, beta, n)


# speedup vs baseline: 1.2157x; 1.0449x over previous
import functools

import jax
import jax.numpy as jnp
from jax import lax
from jax.experimental import pallas as pl
from jax.experimental.pallas import tpu as pltpu
from jax.experimental.pallas import tpu_sc as plsc

NC = 2
NS = 16
L = 16
NW = NC * NS
CHUNK = 128
DEPTH = 4
DC = DEPTH * CHUNK
NPH = 4
D = 128
DH = D // NC

_mesh = functools.partial(
    plsc.VectorSubcoreMesh,
    core_axis_name="c", subcore_axis_name="s", num_cores=NC, num_subcores=NS,
)

_SC_PARAMS = pltpu.CompilerParams(needs_layout_passes=False,
                                  use_tc_tiling_on_sc=False)


def _make_deg_kernel(n_pad, e_w):
    @functools.partial(
        pl.kernel,
        out_type=jax.ShapeDtypeStruct((NW, n_pad), jnp.float32),
        mesh=_mesh(),
        compiler_params=_SC_PARAMS,
        scratch_types=[
            pltpu.VMEM((n_pad,), jnp.float32),
            pltpu.VMEM((e_w,), jnp.int32),
            pltpu.VMEM((e_w,), jnp.float32),
            pltpu.SemaphoreType.DMA,
        ],
    )
    def deg_kernel(dst_hbm, ew_hbm, out_hbm, deg_v, dst_v, ew_v, sem):
        cid = lax.axis_index("c")
        sid = lax.axis_index("s")
        wid = sid * NC + cid
        base = wid * e_w

        ld_d = pltpu.async_copy(dst_hbm.at[pl.ds(base, e_w)], dst_v, sem)
        ld_w = pltpu.async_copy(ew_hbm.at[pl.ds(base, e_w)], ew_v, sem)

        zero = jnp.zeros((L,), jnp.float32)

        def zero_body(i, _):
            deg_v[pl.ds(i * L, L)] = zero
            return 0
        lax.fori_loop(0, n_pad // L, zero_body, 0)
        ld_d.wait()
        ld_w.wait()

        def edge_body(g, _):
            idx = dst_v[pl.ds(g * L, L)]
            val = ew_v[pl.ds(g * L, L)]
            plsc.addupdate_scatter(deg_v, [idx], val)
            return 0
        lax.fori_loop(0, e_w // L, edge_body, 0)

        pltpu.sync_copy(deg_v, out_hbm.at[wid])

    return deg_kernel


def _make_agg_kernel(n_pad, e_w):
    n_chunks = e_w // CHUNK
    n_super = n_chunks // DEPTH
    NB = n_super // NPH
    stripe = n_pad // NS

    scratch = [
        pltpu.VMEM((NPH * DC,), jnp.int32),
        pltpu.VMEM((NPH, DEPTH, CHUNK), jnp.int32),
        pltpu.VMEM((NPH * DC,), jnp.float32),
    ] + [pltpu.VMEM((CHUNK, DH), jnp.float32) for _ in range(DEPTH)
    ] + [
        pltpu.VMEM_SHARED((n_pad, DH), jnp.float32),
        pltpu.VMEM_SHARED((n_pad, DH), jnp.float32),
        pltpu.SemaphoreType.DMA,
    ] + [pltpu.SemaphoreType.DMA for _ in range(NPH)
    ] + [pltpu.SemaphoreType.DMA for _ in range(2 * DEPTH)]

    @functools.partial(
        pl.kernel,
        out_type=jax.ShapeDtypeStruct((NC, n_pad, DH), jnp.float32),
        mesh=_mesh(),
        compiler_params=_SC_PARAMS,
        scratch_types=scratch,
    )
    def agg_kernel(h_hbm, src_hbm, dst_hbm, ew_hbm, out_hbm,
                   src_v, dst_v, ew_v, *rest):
        rows = rest[:DEPTH]
        out_sh = rest[DEPTH]
        h_sh = rest[DEPTH + 1]
        ldsem = rest[DEPTH + 2]
        esem = rest[DEPTH + 3:DEPTH + 3 + NPH]
        gsem = rest[DEPTH + 3 + NPH:DEPTH + 3 + NPH + DEPTH]
        ssem = rest[DEPTH + 3 + NPH + DEPTH:]

        cid = lax.axis_index("c")
        sid = lax.axis_index("s")
        base = sid * e_w
        r0 = sid * stripe

        def _stage_copies(g, q):
            return [
                (src_hbm.at[pl.ds(base + g * DC, DC)],
                 src_v.at[pl.ds(q * DC, DC)]),
                (dst_hbm.at[sid, pl.ds(g * DEPTH, DEPTH)], dst_v.at[q]),
                (ew_hbm.at[pl.ds(base + g * DC, DC)],
                 ew_v.at[pl.ds(q * DC, DC)]),
            ]

        def stage(g, q):
            for s, d in _stage_copies(g, q):
                pltpu.async_copy(s, d, esem[q])

        def stage_wait(g, q):
            for s, d in _stage_copies(g, q):
                pltpu.make_async_copy(s, d, esem[q]).wait()

        def scat_wait(q):
            for k in range(DEPTH):
                pltpu.make_async_copy(
                    rows[k], out_sh.at[dst_v.at[q, k]], ssem[k]).wait()

        ld_h = pltpu.async_copy(h_hbm.at[pl.ds(cid * n_pad + r0, stripe)],
                                h_sh.at[pl.ds(r0, stripe)], ldsem)
        for q in range(NPH):
            stage(q, q)

        zero = jnp.zeros((L,), jnp.float32)

        def zb_body(i, _):
            for j in range(DH // L):
                rows[0][i, pl.ds(j * L, L)] = zero
            return 0
        lax.fori_loop(0, CHUNK, zb_body, 0)

        for i in range(stripe // CHUNK):
            pltpu.sync_copy(rows[0], out_sh.at[pl.ds(r0 + i * CHUNK, CHUNK)])

        ld_h.wait()
        plsc.subcore_barrier()

        def group(g, q):
            gathers = []
            for k in range(DEPTH):
                idx = src_v.at[pl.ds(q * DC + k * CHUNK, CHUNK)]
                gathers.append(
                    pltpu.async_copy(h_sh.at[idx], rows[k], gsem[k]))
            for k in range(DEPTH):
                gathers[k].wait()
                b16 = jnp.full((L,), q * DC + k * CHUNK, jnp.int32)

                @plsc.parallel_loop(0, CHUNK, unroll=4)
                def row_body(r):
                    r16 = lax.broadcast_in_dim(r, (L,), ())
                    nvec = plsc.load_gather(ew_v, [b16 + r16])
                    for j in range(DH // L):
                        rows[k][r, pl.ds(j * L, L)] = (
                            rows[k][r, pl.ds(j * L, L)] * nvec)

                pltpu.make_async_copy(
                    rows[k], out_sh.at[dst_v.at[q, k]], ssem[k]).start(add=True)

        def body(i, first=False, last=False):
            g = NPH * i
            for q in range(NPH):
                if q == 2:
                    plsc.subcore_barrier()
                stage_wait(g + q, q)
                if not (first and q == 0):
                    scat_wait((q - 1) % NPH)
                    if (not first or q > 0) and (not last or q == 0):
                        stage(g + q + NPH - 1, (q - 1) % NPH)
                group(g + q, q)

        body(0, first=True)

        def loop_body(i, _):
            plsc.subcore_barrier()
            body(i)
            return 0
        lax.fori_loop(1, NB - 1, loop_body, 0)

        body(NB - 1, last=True)
        scat_wait(NPH - 1)

        plsc.subcore_barrier()
        pltpu.sync_copy(out_sh.at[pl.ds(r0, stripe)],
                        out_hbm.at[cid, pl.ds(r0, stripe)])

    return agg_kernel


def _tc_matmul(x, w, n_pad):
    n = x.shape[0]

    def mm(x_ref, w_ref, o_ref):
        h = jnp.dot(x_ref[...], w_ref[...], preferred_element_type=jnp.float32)
        o_ref[0, 0:n, :] = h[:, 0:DH]
        o_ref[1, 0:n, :] = h[:, DH:D]
    return pl.pallas_call(
        mm, out_shape=jax.ShapeDtypeStruct((NC, n_pad, DH), jnp.float32))(x, w)


def _tc_dish(deg_p, h1):
    n_pad = deg_p.shape[1]

    def body(dp_ref, h_ref, dis_ref, hs_ref):
        deg = jnp.sum(dp_ref[...], axis=0)
        dis = jnp.where(deg > 0, lax.rsqrt(jnp.maximum(deg, 1e-12)), 0.0)
        dis_ref[0, :] = dis
        hs_ref[0, :, :] = h_ref[0] * dis[:, None]
        hs_ref[1, :, :] = h_ref[1] * dis[:, None]
    return pl.pallas_call(
        body, out_shape=(
            jax.ShapeDtypeStruct((1, n_pad), jnp.float32),
            jax.ShapeDtypeStruct((NC, n_pad, DH), jnp.float32),
        ))(deg_p, h1)


def _tc_mid(parts, dis, b1, w2):
    def body(p_ref, dis_ref, b_ref, w_ref, o_ref):
        d = dis_ref[0][:, None]
        s = jnp.concatenate([p_ref[0], p_ref[1]], axis=-1) * d + b_ref[...]
        h = jnp.dot(jnp.maximum(s, 0.0), w_ref[...],
                    preferred_element_type=jnp.float32)
        o_ref[0, :, :] = h[:, 0:DH] * d
        o_ref[1, :, :] = h[:, DH:D] * d
    n_pad = parts.shape[1]
    return pl.pallas_call(
        body, out_shape=jax.ShapeDtypeStruct((NC, n_pad, DH),
                                             jnp.float32))(parts, dis, b1, w2)


def _tc_final(parts, dis, b2, gamma, beta, n):
    def body(p_ref, dis_ref, b_ref, g_ref, bt_ref, o_ref):
        d = dis_ref[0][0:n, None]
        s = jnp.concatenate([p_ref[0, 0:n, :], p_ref[1, 0:n, :]],
                            axis=-1) * d + b_ref[...]
        mu = jnp.mean(s, axis=-1, keepdims=True)
        var = jnp.mean((s - mu) ** 2, axis=-1, keepdims=True)
        o_ref[...] = ((s - mu) * lax.rsqrt(var + 1e-5) * g_ref[...]
                      + bt_ref[...])
    return pl.pallas_call(
        body, out_shape=jax.ShapeDtypeStruct((n, D), jnp.float32))(
            parts, dis, b2, gamma, beta)


@jax.jit
def kernel(x, edge_index, edge_weight, W1, b1, W2, b2, gamma, beta):
    n = x.shape[0]
    e = edge_weight.shape[0]

    loop = jnp.arange(n, dtype=jnp.int32)
    src = jnp.concatenate([edge_index[0].astype(jnp.int32), loop])
    dst = jnp.concatenate([edge_index[1].astype(jnp.int32), loop])
    ew = jnp.concatenate([edge_weight, jnp.ones((n,), edge_weight.dtype)])

    e_tot = e + n
    grain = NS * CHUNK * DEPTH * NPH
    e_pad = ((e_tot + grain - 1) // grain) * grain
    pad = e_pad - e_tot
    src = jnp.concatenate([src, jnp.zeros((pad,), jnp.int32)])
    dst = jnp.concatenate([dst, jnp.zeros((pad,), jnp.int32)])
    ew = jnp.concatenate([ew, jnp.zeros((pad,), jnp.float32)])
    e_w = e_pad // NS
    e_w_deg = e_pad // NW
    n_chunks = e_w // CHUNK

    ngrain = NS * L
    n_pad = ((n + ngrain - 1) // ngrain) * ngrain

    dst3 = dst.reshape(NS, n_chunks, CHUNK)

    deg_p = _make_deg_kernel(n_pad, e_w_deg)(dst, ew)
    h1 = _tc_matmul(x, W1, n_pad)
    dis, h1s = _tc_dish(deg_p, h1)

    agg = _make_agg_kernel(n_pad, e_w)
    p1 = agg(h1s.reshape(NC * n_pad, DH), src, dst3, ew)
    h2s = _tc_mid(p1, dis, b1, W2)
    p2 = agg(h2s.reshape(NC * n_pad, DH), src, dst3, ew)
    return _tc_final(p2, dis, b2, gamma, beta, n)


# speedup vs baseline: 1.2585x; 1.0352x over previous
import functools

import jax
import jax.numpy as jnp
from jax import lax
from jax.experimental import pallas as pl
from jax.experimental.pallas import tpu as pltpu
from jax.experimental.pallas import tpu_sc as plsc

NC = 2
NS = 16
L = 16
NW = NC * NS
CHUNK = 128
DEPTH = 4
DC = DEPTH * CHUNK
NPH = 4
D = 128
DH = D // NC

_mesh = functools.partial(
    plsc.VectorSubcoreMesh,
    core_axis_name="c", subcore_axis_name="s", num_cores=NC, num_subcores=NS,
)

_SC_PARAMS = pltpu.CompilerParams(needs_layout_passes=False,
                                  use_tc_tiling_on_sc=False)


def _make_deg_kernel(n_pad, e_w):
    @functools.partial(
        pl.kernel,
        out_type=jax.ShapeDtypeStruct((NW, n_pad), jnp.float32),
        mesh=_mesh(),
        compiler_params=_SC_PARAMS,
        scratch_types=[
            pltpu.VMEM((n_pad,), jnp.float32),
            pltpu.VMEM((e_w,), jnp.int32),
            pltpu.VMEM((e_w,), jnp.float32),
            pltpu.SemaphoreType.DMA,
        ],
    )
    def deg_kernel(dst_hbm, ew_hbm, out_hbm, deg_v, dst_v, ew_v, sem):
        cid = lax.axis_index("c")
        sid = lax.axis_index("s")
        wid = sid * NC + cid
        base = wid * e_w

        ld_d = pltpu.async_copy(dst_hbm.at[pl.ds(base, e_w)], dst_v, sem)
        ld_w = pltpu.async_copy(ew_hbm.at[pl.ds(base, e_w)], ew_v, sem)

        zero = jnp.zeros((L,), jnp.float32)

        def zero_body(i, _):
            deg_v[pl.ds(i * L, L)] = zero
            return 0
        lax.fori_loop(0, n_pad // L, zero_body, 0)
        ld_d.wait()
        ld_w.wait()

        def edge_body(g, _):
            idx = dst_v[pl.ds(g * L, L)]
            val = ew_v[pl.ds(g * L, L)]
            plsc.addupdate_scatter(deg_v, [idx], val)
            return 0
        lax.fori_loop(0, e_w // L, edge_body, 0)

        pltpu.sync_copy(deg_v, out_hbm.at[wid])

    return deg_kernel


def _make_agg_kernel(n_pad, e_w):
    n_chunks = e_w // CHUNK
    n_super = n_chunks // DEPTH
    NB = n_super // NPH
    stripe = n_pad // NS

    scratch = [
        pltpu.VMEM((NPH * DC,), jnp.int32),
        pltpu.VMEM((NPH, DEPTH, CHUNK), jnp.int32),
        pltpu.VMEM((NPH * DC,), jnp.float32),
    ] + [pltpu.VMEM((CHUNK, DH), jnp.float32) for _ in range(DEPTH)
    ] + [
        pltpu.VMEM_SHARED((n_pad, DH), jnp.float32),
        pltpu.VMEM_SHARED((n_pad, DH), jnp.float32),
        pltpu.SemaphoreType.DMA,
    ] + [pltpu.SemaphoreType.DMA for _ in range(NPH)
    ] + [pltpu.SemaphoreType.DMA for _ in range(2 * DEPTH)]

    @functools.partial(
        pl.kernel,
        out_type=jax.ShapeDtypeStruct((NC, n_pad, DH), jnp.float32),
        mesh=_mesh(),
        compiler_params=_SC_PARAMS,
        scratch_types=scratch,
    )
    def agg_kernel(h_hbm, src_hbm, dst_hbm, ew_hbm, out_hbm,
                   src_v, dst_v, ew_v, *rest):
        rows = rest[:DEPTH]
        out_sh = rest[DEPTH]
        h_sh = rest[DEPTH + 1]
        ldsem = rest[DEPTH + 2]
        esem = rest[DEPTH + 3:DEPTH + 3 + NPH]
        gsem = rest[DEPTH + 3 + NPH:DEPTH + 3 + NPH + DEPTH]
        ssem = rest[DEPTH + 3 + NPH + DEPTH:]

        cid = lax.axis_index("c")
        sid = lax.axis_index("s")
        base = sid * e_w
        r0 = sid * stripe

        def _stage_copies(g, q):
            return [
                (src_hbm.at[pl.ds(base + g * DC, DC)],
                 src_v.at[pl.ds(q * DC, DC)]),
                (dst_hbm.at[sid, pl.ds(g * DEPTH, DEPTH)], dst_v.at[q]),
                (ew_hbm.at[pl.ds(base + g * DC, DC)],
                 ew_v.at[pl.ds(q * DC, DC)]),
            ]

        def stage(g, q):
            for s, d in _stage_copies(g, q):
                pltpu.async_copy(s, d, esem[q])

        def stage_wait(g, q):
            for s, d in _stage_copies(g, q):
                pltpu.make_async_copy(s, d, esem[q]).wait()

        def scat_wait(q):
            for k in range(DEPTH):
                pltpu.make_async_copy(
                    rows[k], out_sh.at[dst_v.at[q, k]], ssem[k]).wait()

        ld_h = pltpu.async_copy(h_hbm.at[pl.ds(cid * n_pad + r0, stripe)],
                                h_sh.at[pl.ds(r0, stripe)], ldsem)
        for q in range(NPH):
            stage(q, q)

        zero = jnp.zeros((L,), jnp.float32)

        def zb_body(i, _):
            for j in range(DH // L):
                rows[0][i, pl.ds(j * L, L)] = zero
            return 0
        lax.fori_loop(0, CHUNK, zb_body, 0)

        for i in range(stripe // CHUNK):
            pltpu.sync_copy(rows[0], out_sh.at[pl.ds(r0 + i * CHUNK, CHUNK)])

        ld_h.wait()
        plsc.subcore_barrier()

        def group(g, q):
            gathers = []
            for k in range(DEPTH):
                idx = src_v.at[pl.ds(q * DC + k * CHUNK, CHUNK)]
                gathers.append(
                    pltpu.async_copy(h_sh.at[idx], rows[k], gsem[k]))
            for k in range(DEPTH):
                gathers[k].wait()
                b16 = jnp.full((L,), q * DC + k * CHUNK, jnp.int32)

                @plsc.parallel_loop(0, CHUNK, unroll=4)
                def row_body(r):
                    r16 = lax.broadcast_in_dim(r, (L,), ())
                    nvec = plsc.load_gather(ew_v, [b16 + r16])
                    for j in range(DH // L):
                        rows[k][r, pl.ds(j * L, L)] = (
                            rows[k][r, pl.ds(j * L, L)] * nvec)

                pltpu.make_async_copy(
                    rows[k], out_sh.at[dst_v.at[q, k]], ssem[k]).start(add=True)

        def body(i, first=False, last=False):
            g = NPH * i
            for q in range(NPH):
                plsc.subcore_barrier()
                stage_wait(g + q, q)
                if not (first and q == 0):
                    scat_wait((q - 1) % NPH)
                    if (not first or q > 0) and (not last or q == 0):
                        stage(g + q + NPH - 1, (q - 1) % NPH)
                group(g + q, q)

        body(0, first=True)

        def loop_body(i, _):
            body(i)
            return 0
        lax.fori_loop(1, NB - 1, loop_body, 0)

        body(NB - 1, last=True)
        scat_wait(NPH - 1)

        plsc.subcore_barrier()
        pltpu.sync_copy(out_sh.at[pl.ds(r0, stripe)],
                        out_hbm.at[cid, pl.ds(r0, stripe)])

    return agg_kernel


def _tc_matmul(x, w, n_pad):
    n = x.shape[0]

    def mm(x_ref, w_ref, o_ref):
        h = jnp.dot(x_ref[...], w_ref[...], preferred_element_type=jnp.float32)
        o_ref[0, 0:n, :] = h[:, 0:DH]
        o_ref[1, 0:n, :] = h[:, DH:D]
    return pl.pallas_call(
        mm, out_shape=jax.ShapeDtypeStruct((NC, n_pad, DH), jnp.float32))(x, w)


def _tc_dish(deg_p, h1):
    n_pad = deg_p.shape[1]

    def body(dp_ref, h_ref, dis_ref, hs_ref):
        deg = jnp.sum(dp_ref[...], axis=0)
        dis = jnp.where(deg > 0, lax.rsqrt(jnp.maximum(deg, 1e-12)), 0.0)
        dis_ref[0, :] = dis
        hs_ref[0, :, :] = h_ref[0] * dis[:, None]
        hs_ref[1, :, :] = h_ref[1] * dis[:, None]
    return pl.pallas_call(
        body, out_shape=(
            jax.ShapeDtypeStruct((1, n_pad), jnp.float32),
            jax.ShapeDtypeStruct((NC, n_pad, DH), jnp.float32),
        ))(deg_p, h1)


def _tc_mid(parts, dis, b1, w2):
    def body(p_ref, dis_ref, b_ref, w_ref, o_ref):
        d = dis_ref[0][:, None]
        s = jnp.concatenate([p_ref[0], p_ref[1]], axis=-1) * d + b_ref[...]
        h = jnp.dot(jnp.maximum(s, 0.0), w_ref[...],
                    preferred_element_type=jnp.float32)
        o_ref[0, :, :] = h[:, 0:DH] * d
        o_ref[1, :, :] = h[:, DH:D] * d
    n_pad = parts.shape[1]
    return pl.pallas_call(
        body, out_shape=jax.ShapeDtypeStruct((NC, n_pad, DH),
                                             jnp.float32))(parts, dis, b1, w2)


def _tc_final(parts, dis, b2, gamma, beta, n):
    def body(p_ref, dis_ref, b_ref, g_ref, bt_ref, o_ref):
        d = dis_ref[0][0:n, None]
        s = jnp.concatenate([p_ref[0, 0:n, :], p_ref[1, 0:n, :]],
                            axis=-1) * d + b_ref[...]
        mu = jnp.mean(s, axis=-1, keepdims=True)
        var = jnp.mean((s - mu) ** 2, axis=-1, keepdims=True)
        o_ref[...] = ((s - mu) * lax.rsqrt(var + 1e-5) * g_ref[...]
                      + bt_ref[...])
    return pl.pallas_call(
        body, out_shape=jax.ShapeDtypeStruct((n, D), jnp.float32))(
            parts, dis, b2, gamma, beta)


@jax.jit
def kernel(x, edge_index, edge_weight, W1, b1, W2, b2, gamma, beta):
    n = x.shape[0]
    e = edge_weight.shape[0]

    loop = jnp.arange(n, dtype=jnp.int32)
    src = jnp.concatenate([edge_index[0].astype(jnp.int32), loop])
    dst = jnp.concatenate([edge_index[1].astype(jnp.int32), loop])
    ew = jnp.concatenate([edge_weight, jnp.ones((n,), edge_weight.dtype)])

    e_tot = e + n
    grain = NS * CHUNK * DEPTH * NPH
    e_pad = ((e_tot + grain - 1) // grain) * grain
    pad = e_pad - e_tot
    src = jnp.concatenate([src, jnp.zeros((pad,), jnp.int32)])
    dst = jnp.concatenate([dst, jnp.zeros((pad,), jnp.int32)])
    ew = jnp.concatenate([ew, jnp.zeros((pad,), jnp.float32)])
    e_w = e_pad // NS
    e_w_deg = e_pad // NW
    n_chunks = e_w // CHUNK

    ngrain = NS * L
    n_pad = ((n + ngrain - 1) // ngrain) * ngrain

    dst3 = dst.reshape(NS, n_chunks, CHUNK)

    deg_p = _make_deg_kernel(n_pad, e_w_deg)(dst, ew)
    h1 = _tc_matmul(x, W1, n_pad)
    dis, h1s = _tc_dish(deg_p, h1)

    agg = _make_agg_kernel(n_pad, e_w)
    p1 = agg(h1s.reshape(NC * n_pad, DH), src, dst3, ew)
    h2s = _tc_mid(p1, dis, b1, W2)
    p2 = agg(h2s.reshape(NC * n_pad, DH), src, dst3, ew)
    return _tc_final(p2, dis, b2, gamma, beta, n)


# speedup vs baseline: 1.2594x; 1.0007x over previous
import functools

import jax
import jax.numpy as jnp
from jax import lax
from jax.experimental import pallas as pl
from jax.experimental.pallas import tpu as pltpu
from jax.experimental.pallas import tpu_sc as plsc

NC = 2
NS = 16
L = 16
NW = NC * NS
CHUNK = 128
DEPTH = 4
DC = DEPTH * CHUNK
NPH = 4
D = 128
DH = D // NC

_mesh = functools.partial(
    plsc.VectorSubcoreMesh,
    core_axis_name="c", subcore_axis_name="s", num_cores=NC, num_subcores=NS,
)

_SC_PARAMS = pltpu.CompilerParams(needs_layout_passes=False,
                                  use_tc_tiling_on_sc=False)


def _make_deg_kernel(n_pad, e_w):
    @functools.partial(
        pl.kernel,
        out_type=jax.ShapeDtypeStruct((NW, n_pad), jnp.float32),
        mesh=_mesh(),
        compiler_params=_SC_PARAMS,
        scratch_types=[
            pltpu.VMEM((n_pad,), jnp.float32),
            pltpu.VMEM((e_w,), jnp.int32),
            pltpu.VMEM((e_w,), jnp.float32),
            pltpu.SemaphoreType.DMA,
        ],
    )
    def deg_kernel(dst_hbm, ew_hbm, out_hbm, deg_v, dst_v, ew_v, sem):
        cid = lax.axis_index("c")
        sid = lax.axis_index("s")
        wid = sid * NC + cid
        base = wid * e_w

        ld_d = pltpu.async_copy(dst_hbm.at[pl.ds(base, e_w)], dst_v, sem)
        ld_w = pltpu.async_copy(ew_hbm.at[pl.ds(base, e_w)], ew_v, sem)

        zero = jnp.zeros((L,), jnp.float32)

        def zero_body(i, _):
            deg_v[pl.ds(i * L, L)] = zero
            return 0
        lax.fori_loop(0, n_pad // L, zero_body, 0)
        ld_d.wait()
        ld_w.wait()

        def edge_body(g, _):
            idx = dst_v[pl.ds(g * L, L)]
            val = ew_v[pl.ds(g * L, L)]
            plsc.addupdate_scatter(deg_v, [idx], val)
            return 0
        lax.fori_loop(0, e_w // L, edge_body, 0)

        pltpu.sync_copy(deg_v, out_hbm.at[wid])

    return deg_kernel


def _make_agg_kernel(n_pad, e_w):
    n_chunks = e_w // CHUNK
    n_super = n_chunks // DEPTH
    NB = n_super // NPH
    stripe = n_pad // NS

    scratch = [
        pltpu.VMEM((NPH * DC,), jnp.int32),
        pltpu.VMEM((NPH, DEPTH, CHUNK), jnp.int32),
        pltpu.VMEM((NPH * DC,), jnp.float32),
    ] + [pltpu.VMEM((CHUNK, DH), jnp.float32) for _ in range(DEPTH)
    ] + [
        pltpu.VMEM_SHARED((n_pad, DH), jnp.float32),
        pltpu.VMEM_SHARED((n_pad, DH), jnp.float32),
        pltpu.SemaphoreType.DMA,
    ] + [pltpu.SemaphoreType.DMA for _ in range(NPH)
    ] + [pltpu.SemaphoreType.DMA for _ in range(2 * DEPTH)]

    @functools.partial(
        pl.kernel,
        out_type=jax.ShapeDtypeStruct((NC, n_pad, DH), jnp.float32),
        mesh=_mesh(),
        compiler_params=_SC_PARAMS,
        scratch_types=scratch,
    )
    def agg_kernel(h_hbm, src_hbm, dst_hbm, ew_hbm, out_hbm,
                   src_v, dst_v, ew_v, *rest):
        rows = rest[:DEPTH]
        out_sh = rest[DEPTH]
        h_sh = rest[DEPTH + 1]
        ldsem = rest[DEPTH + 2]
        esem = rest[DEPTH + 3:DEPTH + 3 + NPH]
        gsem = rest[DEPTH + 3 + NPH:DEPTH + 3 + NPH + DEPTH]
        ssem = rest[DEPTH + 3 + NPH + DEPTH:]

        cid = lax.axis_index("c")
        sid = lax.axis_index("s")
        base = sid * e_w
        r0 = sid * stripe

        def _stage_copies(g, q):
            return [
                (src_hbm.at[pl.ds(base + g * DC, DC)],
                 src_v.at[pl.ds(q * DC, DC)]),
                (dst_hbm.at[sid, pl.ds(g * DEPTH, DEPTH)], dst_v.at[q]),
                (ew_hbm.at[pl.ds(base + g * DC, DC)],
                 ew_v.at[pl.ds(q * DC, DC)]),
            ]

        def stage(g, q):
            for s, d in _stage_copies(g, q):
                pltpu.async_copy(s, d, esem[q])

        def stage_wait(g, q):
            for s, d in _stage_copies(g, q):
                pltpu.make_async_copy(s, d, esem[q]).wait()

        def scat_wait(q):
            for k in range(DEPTH):
                pltpu.make_async_copy(
                    rows[k], out_sh.at[dst_v.at[q, k]], ssem[k]).wait()

        ld_h = pltpu.async_copy(h_hbm.at[pl.ds(cid * n_pad + r0, stripe)],
                                h_sh.at[pl.ds(r0, stripe)], ldsem)
        for q in range(NPH):
            stage(q, q)

        zero = jnp.zeros((L,), jnp.float32)

        def zb_body(i, _):
            for j in range(DH // L):
                rows[0][i, pl.ds(j * L, L)] = zero
            return 0
        lax.fori_loop(0, CHUNK, zb_body, 0)

        for i in range(stripe // CHUNK):
            pltpu.sync_copy(rows[0], out_sh.at[pl.ds(r0 + i * CHUNK, CHUNK)])

        ld_h.wait()
        plsc.subcore_barrier()

        def group(g, q):
            gathers = []
            for k in range(DEPTH):
                idx = src_v.at[pl.ds(q * DC + k * CHUNK, CHUNK)]
                gathers.append(
                    pltpu.async_copy(h_sh.at[idx], rows[k], gsem[k]))
            for k in range(DEPTH):
                gathers[k].wait()
                plsc.subcore_barrier()
                b16 = jnp.full((L,), q * DC + k * CHUNK, jnp.int32)

                @plsc.parallel_loop(0, CHUNK, unroll=4)
                def row_body(r):
                    r16 = lax.broadcast_in_dim(r, (L,), ())
                    nvec = plsc.load_gather(ew_v, [b16 + r16])
                    for j in range(DH // L):
                        rows[k][r, pl.ds(j * L, L)] = (
                            rows[k][r, pl.ds(j * L, L)] * nvec)

                pltpu.make_async_copy(
                    rows[k], out_sh.at[dst_v.at[q, k]], ssem[k]).start(add=True)

        def body(i, first=False, last=False):
            g = NPH * i
            for q in range(NPH):
                plsc.subcore_barrier()
                stage_wait(g + q, q)
                if not (first and q == 0):
                    scat_wait((q - 1) % NPH)
                    if (not first or q > 0) and (not last or q == 0):
                        stage(g + q + NPH - 1, (q - 1) % NPH)
                group(g + q, q)

        body(0, first=True)

        def loop_body(i, _):
            body(i)
            return 0
        lax.fori_loop(1, NB - 1, loop_body, 0)

        body(NB - 1, last=True)
        scat_wait(NPH - 1)

        plsc.subcore_barrier()
        pltpu.sync_copy(out_sh.at[pl.ds(r0, stripe)],
                        out_hbm.at[cid, pl.ds(r0, stripe)])

    return agg_kernel


def _tc_matmul(x, w, n_pad):
    n = x.shape[0]

    def mm(x_ref, w_ref, o_ref):
        h = jnp.dot(x_ref[...], w_ref[...], preferred_element_type=jnp.float32)
        o_ref[0, 0:n, :] = h[:, 0:DH]
        o_ref[1, 0:n, :] = h[:, DH:D]
    return pl.pallas_call(
        mm, out_shape=jax.ShapeDtypeStruct((NC, n_pad, DH), jnp.float32))(x, w)


def _tc_dish(deg_p, h1):
    n_pad = deg_p.shape[1]

    def body(dp_ref, h_ref, dis_ref, hs_ref):
        deg = jnp.sum(dp_ref[...], axis=0)
        dis = jnp.where(deg > 0, lax.rsqrt(jnp.maximum(deg, 1e-12)), 0.0)
        dis_ref[0, :] = dis
        hs_ref[0, :, :] = h_ref[0] * dis[:, None]
        hs_ref[1, :, :] = h_ref[1] * dis[:, None]
    return pl.pallas_call(
        body, out_shape=(
            jax.ShapeDtypeStruct((1, n_pad), jnp.float32),
            jax.ShapeDtypeStruct((NC, n_pad, DH), jnp.float32),
        ))(deg_p, h1)


def _tc_mid(parts, dis, b1, w2):
    def body(p_ref, dis_ref, b_ref, w_ref, o_ref):
        d = dis_ref[0][:, None]
        s = jnp.concatenate([p_ref[0], p_ref[1]], axis=-1) * d + b_ref[...]
        h = jnp.dot(jnp.maximum(s, 0.0), w_ref[...],
                    preferred_element_type=jnp.float32)
        o_ref[0, :, :] = h[:, 0:DH] * d
        o_ref[1, :, :] = h[:, DH:D] * d
    n_pad = parts.shape[1]
    return pl.pallas_call(
        body, out_shape=jax.ShapeDtypeStruct((NC, n_pad, DH),
                                             jnp.float32))(parts, dis, b1, w2)


def _tc_final(parts, dis, b2, gamma, beta, n):
    def body(p_ref, dis_ref, b_ref, g_ref, bt_ref, o_ref):
        d = dis_ref[0][0:n, None]
        s = jnp.concatenate([p_ref[0, 0:n, :], p_ref[1, 0:n, :]],
                            axis=-1) * d + b_ref[...]
        mu = jnp.mean(s, axis=-1, keepdims=True)
        var = jnp.mean((s - mu) ** 2, axis=-1, keepdims=True)
        o_ref[...] = ((s - mu) * lax.rsqrt(var + 1e-5) * g_ref[...]
                      + bt_ref[...])
    return pl.pallas_call(
        body, out_shape=jax.ShapeDtypeStruct((n, D), jnp.float32))(
            parts, dis, b2, gamma, beta)


@jax.jit
def kernel(x, edge_index, edge_weight, W1, b1, W2, b2, gamma, beta):
    n = x.shape[0]
    e = edge_weight.shape[0]

    loop = jnp.arange(n, dtype=jnp.int32)
    src = jnp.concatenate([edge_index[0].astype(jnp.int32), loop])
    dst = jnp.concatenate([edge_index[1].astype(jnp.int32), loop])
    ew = jnp.concatenate([edge_weight, jnp.ones((n,), edge_weight.dtype)])

    e_tot = e + n
    grain = NS * CHUNK * DEPTH * NPH
    e_pad = ((e_tot + grain - 1) // grain) * grain
    pad = e_pad - e_tot
    src = jnp.concatenate([src, jnp.zeros((pad,), jnp.int32)])
    dst = jnp.concatenate([dst, jnp.zeros((pad,), jnp.int32)])
    ew = jnp.concatenate([ew, jnp.zeros((pad,), jnp.float32)])
    e_w = e_pad // NS
    e_w_deg = e_pad // NW
    n_chunks = e_w // CHUNK

    ngrain = NS * L
    n_pad = ((n + ngrain - 1) // ngrain) * ngrain

    dst3 = dst.reshape(NS, n_chunks, CHUNK)

    deg_p = _make_deg_kernel(n_pad, e_w_deg)(dst, ew)
    h1 = _tc_matmul(x, W1, n_pad)
    dis, h1s = _tc_dish(deg_p, h1)

    agg = _make_agg_kernel(n_pad, e_w)
    p1 = agg(h1s.reshape(NC * n_pad, DH), src, dst3, ew)
    h2s = _tc_mid(p1, dis, b1, W2)
    p2 = agg(h2s.reshape(NC * n_pad, DH), src, dst3, ew)
    return _tc_final(p2, dis, b2, gamma, beta, n)
